# Initial kernel scaffold; baseline (speedup 1.0000x reference)
import jax
import jax.numpy as jnp
from jax.experimental import pallas as pl


def kernel(feature_index, embed_index, scaling, W):
    raise NotImplementedError("write your pallas kernel here")


# speedup vs baseline: 2.6176x; 2.6176x over previous
import functools

import jax
import jax.numpy as jnp
from jax import lax
from jax.experimental import pallas as pl
from jax.experimental.pallas import tpu as pltpu
from jax.experimental.pallas import tpu_sc as plsc

DIM = 128
LANES = 16
NW = 32
K = 128
GPC = K // LANES


def _make_kernel(batch):
    rows_w = batch // NW
    mesh = plsc.VectorSubcoreMesh(core_axis_name="c", subcore_axis_name="s")

    @functools.partial(
        pl.kernel,
        out_type=jax.ShapeDtypeStruct((batch * DIM,), jnp.float32),
        mesh=mesh,
        scratch_types=[
            pltpu.VMEM((48,), jnp.int32),
            pltpu.VMEM((2, K), jnp.int32),
            pltpu.VMEM((2, K), jnp.int32),
            pltpu.VMEM((2, K), jnp.float32),
            pltpu.VMEM((2, K, DIM), jnp.float32),
            pltpu.VMEM((rows_w * DIM,), jnp.float32),
            pltpu.SemaphoreType.DMA,
            pltpu.SemaphoreType.DMA,
            pltpu.SemaphoreType.DMA,
            pltpu.SemaphoreType.DMA,
        ],
        compiler_params=pltpu.CompilerParams(needs_layout_passes=False),
    )
    def ker(fi_hbm, ei_hbm, sc_hbm, w_hbm, starts_hbm, out_hbm,
            starts_v, fidx_v, ei_v, scl_v, rows_v, acc_v,
            sem_i0, sem_i1, sem_g0, sem_g1):
        sem_i = (sem_i0, sem_i1)
        sem_g = (sem_g0, sem_g1)
        cid = lax.axis_index("c")
        sid = lax.axis_index("s")
        wid = sid * 2 + cid
        base = wid * rows_w

        pltpu.sync_copy(starts_hbm, starts_v)
        svec = starts_v[pl.ds(wid, LANES)]
        lo = svec[0]
        hi = svec[1]
        lo_a = lo - lax.rem(lo, 8)
        nchunks = (hi - lo_a + K - 1) // K
        npairs = (nchunks + 1) // 2

        zeros = jnp.zeros((LANES,), jnp.float32)

        def zero_body(i, _):
            acc_v[pl.ds(i * LANES, LANES)] = zeros
            return 0

        lax.fori_loop(0, rows_w * DIM // LANES, zero_body, 0)

        iota = lax.broadcasted_iota(jnp.int32, (LANES,), 0)
        tsel = [jnp.full((LANES,), t, jnp.int32) for t in range(LANES)]

        def chunk_off(m):
            return pl.multiple_of(lo_a + m * K, 8)

        def fire_idx(m, p):
            off = chunk_off(m)
            pltpu.async_copy(fi_hbm.at[pl.ds(off, K)], fidx_v.at[p], sem_i[p])
            pltpu.async_copy(ei_hbm.at[pl.ds(off, K)], ei_v.at[p], sem_i[p])
            pltpu.async_copy(sc_hbm.at[pl.ds(off, K)], scl_v.at[p], sem_i[p])

        def wait_idx(p):
            pltpu.make_async_copy(fi_hbm.at[pl.ds(0, K)], fidx_v.at[p], sem_i[p]).wait()
            pltpu.make_async_copy(ei_hbm.at[pl.ds(0, K)], ei_v.at[p], sem_i[p]).wait()
            pltpu.make_async_copy(sc_hbm.at[pl.ds(0, K)], scl_v.at[p], sem_i[p]).wait()

        def fire_gather(p):
            pltpu.async_copy(w_hbm.at[fidx_v.at[p]], rows_v.at[p], sem_g[p])

        def wait_gather(p):
            pltpu.make_async_copy(w_hbm.at[fidx_v.at[p]], rows_v.at[p], sem_g[p]).wait()

        def process(m, p):
            off = chunk_off(m)

            def group_body(g, _):
                sl = pl.ds(g * LANES, LANES)
                jg = off + g * LANES + iota
                e = ei_v[p, sl]
                ls16 = jnp.clip(e - base, 0, rows_w - 1) * DIM
                s_raw = scl_v[p, sl]
                s16 = jnp.where((jg >= lo) & (jg < hi), s_raw, 0.0)
                for t in range(LANES):
                    j = g * LANES + t
                    a = jnp.take_along_axis(ls16, tsel[t], axis=0) + iota
                    s = jnp.take_along_axis(s16, tsel[t], axis=0)
                    for h in range(DIM // LANES):
                        v = rows_v[p, j, pl.ds(h * LANES, LANES)]
                        plsc.addupdate_scatter(acc_v, [a + h * LANES], v * s)
                return 0

            lax.fori_loop(0, GPC, group_body, 0)

        fire_idx(0, 0)
        fire_idx(1, 1)
        wait_idx(0)
        fire_gather(0)

        def half_iter(m, p):
            q = 1 - p
            wait_idx(q)
            fire_gather(q)
            wait_gather(p)
            process(m, p)
            fire_idx(m + 2, p)

        def pair_body(m2, _):
            m = 2 * m2
            half_iter(m, 0)
            half_iter(m + 1, 1)
            return 0

        lax.fori_loop(0, npairs, pair_body, 0)

        wait_gather(0)
        wait_idx(1)

        pltpu.sync_copy(acc_v, out_hbm.at[pl.ds(base * DIM, rows_w * DIM)])

    return ker


def kernel(feature_index, embed_index, scaling, W):
    batch = 16384
    rows_w = batch // NW
    bounds = jnp.arange(NW + 1, dtype=jnp.int32) * rows_w
    starts = jnp.searchsorted(embed_index, bounds).astype(jnp.int32)
    starts = jnp.pad(starts, (0, 48 - (NW + 1)))
    pad = 4 * K
    fi = jnp.pad(feature_index, (0, pad))
    ei = jnp.pad(embed_index, (0, pad))
    sc = jnp.pad(scaling, (0, pad))
    ker = _make_kernel(batch)
    return ker(fi, ei, sc, W, starts).reshape(batch, DIM)


# speedup vs baseline: 6.5677x; 2.5090x over previous
import functools

import jax
import jax.numpy as jnp
from jax import lax
from jax.experimental import pallas as pl
from jax.experimental.pallas import tpu as pltpu
from jax.experimental.pallas import tpu_sc as plsc

DIM = 128
LANES = 16
NW = 32
K = 128
GPC = K // LANES


def _make_kernel(batch):
    rows_w = batch // NW
    mesh = plsc.VectorSubcoreMesh(core_axis_name="c", subcore_axis_name="s")

    @functools.partial(
        pl.kernel,
        out_type=jax.ShapeDtypeStruct((batch * DIM,), jnp.float32),
        mesh=mesh,
        scratch_types=[
            pltpu.VMEM((48,), jnp.int32),
            pltpu.VMEM((2, K), jnp.int32),
            pltpu.VMEM((2, K), jnp.int32),
            pltpu.VMEM((2, K), jnp.float32),
            pltpu.VMEM((2, K, DIM), jnp.float32),
            pltpu.VMEM((rows_w * DIM,), jnp.float32),
            pltpu.SemaphoreType.DMA,
            pltpu.SemaphoreType.DMA,
            pltpu.SemaphoreType.DMA,
            pltpu.SemaphoreType.DMA,
        ],
        compiler_params=pltpu.CompilerParams(needs_layout_passes=False),
    )
    def ker(fi_hbm, ei_hbm, sc_hbm, w_hbm, starts_hbm, out_hbm,
            starts_v, fidx_v, ei_v, scl_v, rows_v, acc_v,
            sem_i0, sem_i1, sem_g0, sem_g1):
        sem_i = (sem_i0, sem_i1)
        sem_g = (sem_g0, sem_g1)
        cid = lax.axis_index("c")
        sid = lax.axis_index("s")
        wid = sid * 2 + cid
        base = wid * rows_w

        pltpu.sync_copy(starts_hbm, starts_v)
        svec = starts_v[pl.ds(wid, LANES)]
        lo = svec[0]
        hi = svec[1]
        lo_a = lo - lax.rem(lo, 8)
        nchunks = (hi - lo_a + K - 1) // K
        npairs = (nchunks + 1) // 2

        zeros = jnp.zeros((LANES,), jnp.float32)

        @plsc.parallel_loop(0, rows_w * DIM // LANES, unroll=8)
        def _zero(i):
            acc_v[pl.ds(i * LANES, LANES)] = zeros

        iota = lax.broadcasted_iota(jnp.int32, (LANES,), 0)

        def chunk_off(m):
            return pl.multiple_of(lo_a + m * K, 8)

        def fire_idx(m, p):
            off = chunk_off(m)
            pltpu.async_copy(fi_hbm.at[pl.ds(off, K)], fidx_v.at[p], sem_i[p])
            pltpu.async_copy(ei_hbm.at[pl.ds(off, K)], ei_v.at[p], sem_i[p])
            pltpu.async_copy(sc_hbm.at[pl.ds(off, K)], scl_v.at[p], sem_i[p])

        def wait_idx(p):
            pltpu.make_async_copy(fi_hbm.at[pl.ds(0, K)], fidx_v.at[p], sem_i[p]).wait()
            pltpu.make_async_copy(ei_hbm.at[pl.ds(0, K)], ei_v.at[p], sem_i[p]).wait()
            pltpu.make_async_copy(sc_hbm.at[pl.ds(0, K)], scl_v.at[p], sem_i[p]).wait()

        def fire_gather(p):
            pltpu.async_copy(w_hbm.at[fidx_v.at[p]], rows_v.at[p], sem_g[p])

        def wait_gather(p):
            pltpu.make_async_copy(w_hbm.at[fidx_v.at[p]], rows_v.at[p], sem_g[p]).wait()

        def process(m, p):
            off = chunk_off(m)

            @plsc.parallel_loop(0, GPC, unroll=2)
            def _stage(g):
                sl = pl.ds(g * LANES, LANES)
                jg = off + g * LANES + iota
                e = ei_v[p, sl]
                ei_v[p, sl] = jnp.clip(e - base, 0, rows_w - 1) * DIM
                s_raw = scl_v[p, sl]
                scl_v[p, sl] = jnp.where((jg >= lo) & (jg < hi), s_raw, 0.0)

            @plsc.parallel_loop(0, K, unroll=4)
            def _row(j):
                jf = jnp.full((LANES,), 0, jnp.int32) + j
                a = plsc.load_gather(ei_v.at[p], [jf]) + iota
                s = plsc.load_gather(scl_v.at[p], [jf])
                vs = [rows_v[p, j, pl.ds(h * LANES, LANES)]
                      for h in range(DIM // LANES)]
                for h in range(DIM // LANES):
                    plsc.addupdate_scatter(acc_v, [a + h * LANES], vs[h] * s)

        fire_idx(0, 0)
        fire_idx(1, 1)
        wait_idx(0)
        fire_gather(0)

        def half_iter(m, p):
            q = 1 - p
            wait_idx(q)
            fire_gather(q)
            wait_gather(p)
            process(m, p)
            fire_idx(m + 2, p)

        def pair_body(m2, _):
            m = 2 * m2
            half_iter(m, 0)
            half_iter(m + 1, 1)
            return 0

        lax.fori_loop(0, npairs, pair_body, 0)

        wait_gather(0)
        wait_idx(1)

        pltpu.sync_copy(acc_v, out_hbm.at[pl.ds(base * DIM, rows_w * DIM)])

    return ker


def kernel(feature_index, embed_index, scaling, W):
    batch = 16384
    rows_w = batch // NW
    bounds = jnp.arange(NW + 1, dtype=jnp.int32) * rows_w
    starts = jnp.searchsorted(embed_index, bounds).astype(jnp.int32)
    starts = jnp.pad(starts, (0, 48 - (NW + 1)))
    pad = 4 * K
    fi = jnp.pad(feature_index, (0, pad))
    ei = jnp.pad(embed_index, (0, pad))
    sc = jnp.pad(scaling, (0, pad))
    ker = _make_kernel(batch)
    return ker(fi, ei, sc, W, starts).reshape(batch, DIM)


# speedup vs baseline: 6.6128x; 1.0069x over previous
import functools

import jax
import jax.numpy as jnp
from jax import lax
from jax.experimental import pallas as pl
from jax.experimental.pallas import tpu as pltpu
from jax.experimental.pallas import tpu_sc as plsc

DIM = 128
LANES = 16
NW = 32
K = 128
GPC = K // LANES


def _make_kernel(batch):
    rows_w = batch // NW
    mesh = plsc.VectorSubcoreMesh(core_axis_name="c", subcore_axis_name="s")

    @functools.partial(
        pl.kernel,
        out_type=jax.ShapeDtypeStruct((batch * DIM,), jnp.float32),
        mesh=mesh,
        scratch_types=[
            pltpu.VMEM((48,), jnp.int32),
            pltpu.VMEM((2, K), jnp.int32),
            pltpu.VMEM((2, K), jnp.int32),
            pltpu.VMEM((2, K), jnp.float32),
            pltpu.VMEM((2, K, DIM), jnp.float32),
            pltpu.VMEM((rows_w * DIM,), jnp.float32),
            pltpu.SemaphoreType.DMA,
            pltpu.SemaphoreType.DMA,
            pltpu.SemaphoreType.DMA,
            pltpu.SemaphoreType.DMA,
        ],
        compiler_params=pltpu.CompilerParams(needs_layout_passes=False),
    )
    def ker(fi_hbm, ei_hbm, sc_hbm, w_hbm, starts_hbm, out_hbm,
            starts_v, fidx_v, ei_v, scl_v, rows_v, acc_v,
            sem_i0, sem_i1, sem_g0, sem_g1):
        sem_i = (sem_i0, sem_i1)
        sem_g = (sem_g0, sem_g1)
        cid = lax.axis_index("c")
        sid = lax.axis_index("s")
        wid = sid * 2 + cid
        base = wid * rows_w

        pltpu.sync_copy(starts_hbm, starts_v)
        svec = starts_v[pl.ds(wid, LANES)]
        lo = svec[0]
        hi = svec[1]
        lo_a = lo - lax.rem(lo, 8)
        nchunks = (hi - lo_a + K - 1) // K
        npairs = (nchunks + 1) // 2

        zeros = jnp.zeros((LANES,), jnp.float32)

        @plsc.parallel_loop(0, rows_w * DIM // LANES, unroll=8)
        def _zero(i):
            acc_v[pl.ds(i * LANES, LANES)] = zeros

        iota = lax.broadcasted_iota(jnp.int32, (LANES,), 0)

        def chunk_off(m):
            return pl.multiple_of(lo_a + m * K, 8)

        def fire_idx(m, p):
            off = chunk_off(m)
            pltpu.async_copy(fi_hbm.at[pl.ds(off, K)], fidx_v.at[p], sem_i[p])
            pltpu.async_copy(ei_hbm.at[pl.ds(off, K)], ei_v.at[p], sem_i[p])
            pltpu.async_copy(sc_hbm.at[pl.ds(off, K)], scl_v.at[p], sem_i[p])

        def wait_idx(p):
            pltpu.make_async_copy(fi_hbm.at[pl.ds(0, K)], fidx_v.at[p], sem_i[p]).wait()
            pltpu.make_async_copy(ei_hbm.at[pl.ds(0, K)], ei_v.at[p], sem_i[p]).wait()
            pltpu.make_async_copy(sc_hbm.at[pl.ds(0, K)], scl_v.at[p], sem_i[p]).wait()

        NSUB = 4
        SUB = K // NSUB

        def fire_gather(p):
            for t in range(NSUB):
                pltpu.async_copy(
                    w_hbm.at[fidx_v.at[p, pl.ds(t * SUB, SUB)]],
                    rows_v.at[p, pl.ds(t * SUB, SUB)], sem_g[p])

        def wait_gather(p):
            for t in range(NSUB):
                pltpu.make_async_copy(
                    w_hbm.at[fidx_v.at[p, pl.ds(t * SUB, SUB)]],
                    rows_v.at[p, pl.ds(t * SUB, SUB)], sem_g[p]).wait()

        def process(m, p):
            off = chunk_off(m)

            @plsc.parallel_loop(0, GPC, unroll=2)
            def _stage(g):
                sl = pl.ds(g * LANES, LANES)
                jg = off + g * LANES + iota
                e = ei_v[p, sl]
                ei_v[p, sl] = jnp.clip(e - base, 0, rows_w - 1) * DIM
                s_raw = scl_v[p, sl]
                scl_v[p, sl] = jnp.where((jg >= lo) & (jg < hi), s_raw, 0.0)

            def flush(a_prev, acc):
                for h in range(DIM // LANES):
                    view = acc_v.at[pl.ds(h * LANES,
                                          rows_w * DIM - h * LANES)]
                    plsc.addupdate_scatter(view, [a_prev], acc[h])

            init = (iota, jnp.int32(-1)) + \
                (jnp.zeros((LANES,), jnp.float32),) * (DIM // LANES)

            @plsc.parallel_loop(0, K, unroll=2, carry=init)
            def _row(j, carry):
                a_prev, s_prev = carry[0], carry[1]
                acc = carry[2:]
                gbase = (j // LANES) * LANES
                lane = jnp.full((LANES,), 0, jnp.int32) + (j - gbase)
                a16 = ei_v[p, pl.ds(gbase, LANES)]
                s16 = scl_v[p, pl.ds(gbase, LANES)]
                av = jnp.take_along_axis(a16, lane, axis=0) + iota
                sv = jnp.take_along_axis(s16, lane, axis=0)
                prod = tuple(rows_v[p, j, pl.ds(h * LANES, LANES)] * sv
                             for h in range(DIM // LANES))
                s_cur = av[0]

                def if_same(_):
                    return tuple(acc[h] + prod[h]
                                 for h in range(DIM // LANES))

                def if_new(_):
                    flush(a_prev, acc)
                    return prod

                newacc = lax.cond(s_cur == s_prev, if_same, if_new, 0)
                return (av, s_cur) + newacc

            flush(_row[0], _row[2:])

        fire_idx(0, 0)
        fire_idx(1, 1)
        wait_idx(0)
        fire_gather(0)

        def half_iter(m, p):
            q = 1 - p
            wait_idx(q)
            fire_gather(q)
            wait_gather(p)
            process(m, p)
            fire_idx(m + 2, p)

        def pair_body(m2, _):
            m = 2 * m2
            half_iter(m, 0)
            half_iter(m + 1, 1)
            return 0

        lax.fori_loop(0, npairs, pair_body, 0)

        wait_gather(0)
        wait_idx(1)

        pltpu.sync_copy(acc_v, out_hbm.at[pl.ds(base * DIM, rows_w * DIM)])

    return ker


def kernel(feature_index, embed_index, scaling, W):
    batch = 16384
    rows_w = batch // NW
    bounds = jnp.arange(NW + 1, dtype=jnp.int32) * rows_w
    starts = jnp.searchsorted(embed_index, bounds).astype(jnp.int32)
    starts = jnp.pad(starts, (0, 48 - (NW + 1)))
    pad = 4 * K
    fi = jnp.pad(feature_index, (0, pad))
    ei = jnp.pad(embed_index, (0, pad))
    sc = jnp.pad(scaling, (0, pad))
    ker = _make_kernel(batch)
    return ker(fi, ei, sc, W, starts).reshape(batch, DIM)
